# Initial kernel scaffold; baseline (speedup 1.0000x reference)
import jax
import jax.numpy as jnp
from jax.experimental import pallas as pl


def kernel(x, edge_index, batch, enc_W, enc_b, W1, b1, g1, be1, W2, b2, g2, be2, hW1, hb1, hg1, hbe1, hW2, hb2, hg2, hbe2, hW3, hb3):
    raise NotImplementedError("write your pallas kernel here")


# speedup vs baseline: 3.5600x; 3.5600x over previous
import functools

import jax
import jax.numpy as jnp
from jax import lax
from jax.experimental import pallas as pl
from jax.experimental.pallas import tpu as pltpu
from jax.experimental.pallas import tpu_sc as plsc

N = 10000
E = 160000
F_IN = 256
H = 512
LAYERS = 8
G = 128
T_OUT = 5

NCHUNK = 4
CW = 128
RB = 1000
GRID = N // RB
EPS = 1e-5

NTILE = 16
EB = 80
EPT = E // NTILE
NBATCH = EPT // EB
RPT = 624
RTAIL = N - NTILE * RPT

_f32 = jnp.float32
_bf16 = jnp.bfloat16


def _dot16(a, w):
    return jnp.dot(a.astype(_bf16), w.astype(_bf16),
                   preferred_element_type=_f32)


def _sc_agg_body(hc_hbm, src_hbm, dst_hbm, u_hbm, srcv, dstv, rows, spm, sem):
    c = lax.axis_index("c")
    s = lax.axis_index("s")
    pltpu.sync_copy(src_hbm.at[s], srcv)
    pltpu.sync_copy(dst_hbm.at[s], dstv)
    for j in range(2):
        chunk = c * 2 + j
        pltpu.sync_copy(hc_hbm.at[chunk, pl.ds(s * RPT, RPT)],
                        spm.at[pl.ds(s * RPT, RPT)])

        @pl.when(s == NTILE - 1)
        def _():
            pltpu.sync_copy(hc_hbm.at[chunk, pl.ds(NTILE * RPT, RTAIL)],
                            spm.at[pl.ds(NTILE * RPT, RTAIL)])

        plsc.subcore_barrier()

        def body(b, carry):
            pltpu.async_copy(hc_hbm.at[chunk].at[srcv.at[b]], rows, sem).wait()
            pltpu.sync_copy(rows, spm.at[dstv.at[b]], add=True)
            return carry

        lax.fori_loop(0, NBATCH, body, 0)
        plsc.subcore_barrier()
        pltpu.sync_copy(spm.at[pl.ds(s * RPT, RPT)],
                        u_hbm.at[chunk, pl.ds(s * RPT, RPT)])

        @pl.when(s == NTILE - 1)
        def _():
            pltpu.sync_copy(spm.at[pl.ds(NTILE * RPT, RTAIL)],
                            u_hbm.at[chunk, pl.ds(NTILE * RPT, RTAIL)])

        plsc.subcore_barrier()


@functools.lru_cache(maxsize=1)
def _sc_agg_kernel():
    return pl.kernel(
        _sc_agg_body,
        out_type=jax.ShapeDtypeStruct((NCHUNK, N, CW), _f32),
        mesh=plsc.VectorSubcoreMesh(core_axis_name="c", subcore_axis_name="s"),
        scratch_types=[
            pltpu.VMEM((NBATCH, EB), jnp.int32),
            pltpu.VMEM((NBATCH, EB), jnp.int32),
            pltpu.VMEM((EB, CW), _f32),
            pltpu.VMEM_SHARED((N, CW), _f32),
            pltpu.SemaphoreType.DMA,
        ],
    )


def _sc_agg(hc, src3, dst3):
    return _sc_agg_kernel()(hc, src3, dst3)


def _enc_body(x_ref, w_ref, b_ref, hc_ref):
    h = _dot16(x_ref[...], w_ref[...]) + b_ref[...]
    for ch in range(NCHUNK):
        hc_ref[ch, :, :] = h[:, ch * CW:(ch + 1) * CW]


def _enc(x, w, b):
    return pl.pallas_call(
        _enc_body,
        grid=(GRID,),
        in_specs=[
            pl.BlockSpec((RB, F_IN), lambda i: (i, 0)),
            pl.BlockSpec((F_IN, H), lambda i: (0, 0)),
            pl.BlockSpec((1, H), lambda i: (0, 0)),
        ],
        out_specs=pl.BlockSpec((NCHUNK, RB, CW), lambda i: (0, i, 0)),
        out_shape=jax.ShapeDtypeStruct((NCHUNK, N, CW), _f32),
    )(x, w, b)


def _accum_stats(st_ref, z, i):
    s0 = jnp.sum(z, axis=0, keepdims=True)
    s1 = jnp.sum(z * z, axis=0, keepdims=True)
    st = jnp.concatenate([s0, s1], axis=0)

    @pl.when(i == 0)
    def _():
        st_ref[...] = st

    @pl.when(i > 0)
    def _():
        st_ref[...] = st_ref[...] + st


def _scale_shift(st_ref, g_ref, be_ref):
    mean = st_ref[0:1, :] / N
    var = st_ref[1:2, :] / N - mean * mean
    scale = g_ref[...] * lax.rsqrt(var + EPS)
    shift = be_ref[...] - mean * scale
    return scale, shift


def _mlp1_body(u_ref, w_ref, b_ref, z_ref, st_ref):
    i = pl.program_id(0)
    acc = jnp.zeros((RB, 2 * H), _f32)
    for ch in range(NCHUNK):
        acc += _dot16(u_ref[ch, :, :], w_ref[ch, :, :])
    z = acc + b_ref[...]
    z_ref[...] = z
    _accum_stats(st_ref, z, i)


def _mlp1(u, w, b):
    return pl.pallas_call(
        _mlp1_body,
        grid=(GRID,),
        in_specs=[
            pl.BlockSpec((NCHUNK, RB, CW), lambda i: (0, i, 0)),
            pl.BlockSpec((NCHUNK, CW, 2 * H), lambda i: (0, 0, 0)),
            pl.BlockSpec((1, 2 * H), lambda i: (0, 0)),
        ],
        out_specs=[
            pl.BlockSpec((RB, 2 * H), lambda i: (i, 0)),
            pl.BlockSpec((2, 2 * H), lambda i: (0, 0)),
        ],
        out_shape=[
            jax.ShapeDtypeStruct((N, 2 * H), _f32),
            jax.ShapeDtypeStruct((2, 2 * H), _f32),
        ],
    )(u, w, b)


def _mlp2_body(z_ref, st_ref, g_ref, be_ref, w_ref, b_ref, z2_ref, st2_ref):
    i = pl.program_id(0)
    scale, shift = _scale_shift(st_ref, g_ref, be_ref)
    t = z_ref[...] * scale + shift
    t = t * jax.nn.sigmoid(t)
    z2 = _dot16(t, w_ref[...]) + b_ref[...]
    z2_ref[...] = z2
    _accum_stats(st2_ref, z2, i)


def _mlp2(z, st, g, be, w, b):
    return pl.pallas_call(
        _mlp2_body,
        grid=(GRID,),
        in_specs=[
            pl.BlockSpec((RB, 2 * H), lambda i: (i, 0)),
            pl.BlockSpec((2, 2 * H), lambda i: (0, 0)),
            pl.BlockSpec((1, 2 * H), lambda i: (0, 0)),
            pl.BlockSpec((1, 2 * H), lambda i: (0, 0)),
            pl.BlockSpec((2 * H, H), lambda i: (0, 0)),
            pl.BlockSpec((1, H), lambda i: (0, 0)),
        ],
        out_specs=[
            pl.BlockSpec((RB, H), lambda i: (i, 0)),
            pl.BlockSpec((2, H), lambda i: (0, 0)),
        ],
        out_shape=[
            jax.ShapeDtypeStruct((N, H), _f32),
            jax.ShapeDtypeStruct((2, H), _f32),
        ],
    )(z, st, g, be, w, b)


def _resid_body(z2_ref, st_ref, g_ref, be_ref, h_ref, out_ref):
    scale, shift = _scale_shift(st_ref, g_ref, be_ref)
    t = z2_ref[...] * scale + shift
    t = t * jax.nn.sigmoid(t)
    for ch in range(NCHUNK):
        out_ref[ch, :, :] = h_ref[ch, :, :] + t[:, ch * CW:(ch + 1) * CW]


def _resid0_body(z2_ref, st_ref, g_ref, be_ref, out_ref):
    scale, shift = _scale_shift(st_ref, g_ref, be_ref)
    t = z2_ref[...] * scale + shift
    t = t * jax.nn.sigmoid(t)
    for ch in range(NCHUNK):
        out_ref[ch, :, :] = t[:, ch * CW:(ch + 1) * CW]


def _resid(z2, st, g, be, hc):
    return pl.pallas_call(
        _resid_body,
        grid=(GRID,),
        in_specs=[
            pl.BlockSpec((RB, H), lambda i: (i, 0)),
            pl.BlockSpec((2, H), lambda i: (0, 0)),
            pl.BlockSpec((1, H), lambda i: (0, 0)),
            pl.BlockSpec((1, H), lambda i: (0, 0)),
            pl.BlockSpec((NCHUNK, RB, CW), lambda i: (0, i, 0)),
        ],
        out_specs=pl.BlockSpec((NCHUNK, RB, CW), lambda i: (0, i, 0)),
        out_shape=jax.ShapeDtypeStruct((NCHUNK, N, CW), _f32),
    )(z2, st, g, be, hc)


def _resid0(z2, st, g, be):
    return pl.pallas_call(
        _resid0_body,
        grid=(GRID,),
        in_specs=[
            pl.BlockSpec((RB, H), lambda i: (i, 0)),
            pl.BlockSpec((2, H), lambda i: (0, 0)),
            pl.BlockSpec((1, H), lambda i: (0, 0)),
            pl.BlockSpec((1, H), lambda i: (0, 0)),
        ],
        out_specs=pl.BlockSpec((NCHUNK, RB, CW), lambda i: (0, i, 0)),
        out_shape=jax.ShapeDtypeStruct((NCHUNK, N, CW), _f32),
    )(z2, st, g, be)


def _pool_body(hc_ref, b3_ref, ps_ref, pc_ref):
    i = pl.program_id(0)
    bid = b3_ref[0, 0, :]
    oh = (bid[:, None] == lax.broadcasted_iota(jnp.int32, (RB, G), 1))
    oh = oh.astype(_f32)
    hfull = jnp.concatenate([hc_ref[ch, :, :] for ch in range(NCHUNK)], axis=1)
    ps = lax.dot_general(oh, hfull, (((0,), (0,)), ((), ())),
                         preferred_element_type=_f32, precision=lax.Precision.HIGHEST)
    pc = lax.dot_general(oh, jnp.ones((RB, 8), _f32), (((0,), (0,)), ((), ())),
                         preferred_element_type=_f32, precision=lax.Precision.HIGHEST)

    @pl.when(i == 0)
    def _():
        ps_ref[...] = ps
        pc_ref[...] = pc

    @pl.when(i > 0)
    def _():
        ps_ref[...] = ps_ref[...] + ps
        pc_ref[...] = pc_ref[...] + pc


def _pool(hc, batch3):
    return pl.pallas_call(
        _pool_body,
        grid=(GRID,),
        in_specs=[
            pl.BlockSpec((NCHUNK, RB, CW), lambda i: (0, i, 0)),
            pl.BlockSpec((1, 1, RB), lambda i: (i, 0, 0)),
        ],
        out_specs=[
            pl.BlockSpec((G, H), lambda i: (0, 0)),
            pl.BlockSpec((G, 8), lambda i: (0, 0)),
        ],
        out_shape=[
            jax.ShapeDtypeStruct((G, H), _f32),
            jax.ShapeDtypeStruct((G, 8), _f32),
        ],
    )(hc, batch3)


def _head_body(ps_ref, pc_ref, w1, b1, g1, be1, w2, b2, g2, be2, w3, b3,
               out_ref):
    def bn_silu(o, g, be):
        m = jnp.mean(o, axis=0, keepdims=True)
        v = jnp.mean((o - m) * (o - m), axis=0, keepdims=True)
        o = (o - m) * lax.rsqrt(v + EPS) * g[...] + be[...]
        return o * jax.nn.sigmoid(o)

    cnt = pc_ref[:, 0:1]
    pooled = ps_ref[...] / jnp.maximum(cnt, 1.0)
    o = _dot16(pooled, w1[...]) + b1[...]
    o = bn_silu(o, g1, be1)
    o = _dot16(o, w2[...]) + b2[...]
    o = bn_silu(o, g2, be2)
    out_ref[...] = _dot16(o, w3[...]) + b3[...]


def _head(ps, pc, w1, b1, g1, be1, w2, b2, g2, be2, w3, b3):
    specs = [pl.BlockSpec(a.shape, lambda: tuple(0 for _ in a.shape))
             for a in (ps, pc, w1, b1, g1, be1, w2, b2, g2, be2, w3, b3)]
    return pl.pallas_call(
        _head_body,
        in_specs=specs,
        out_specs=pl.BlockSpec((G, T_OUT), lambda: (0, 0)),
        out_shape=jax.ShapeDtypeStruct((G, T_OUT), _f32),
    )(ps, pc, w1, b1, g1, be1, w2, b2, g2, be2, w3, b3)


def kernel(x, edge_index, batch, enc_W, enc_b, W1, b1, g1, be1, W2, b2, g2,
           be2, hW1, hb1, hg1, hbe1, hW2, hb2, hg2, hbe2, hW3, hb3):
    src3 = edge_index[0].reshape(NTILE, NBATCH, EB)
    dst3 = edge_index[1].reshape(NTILE, NBATCH, EB)
    batch3 = batch.reshape(GRID, 1, RB)
    W1c = W1.reshape(LAYERS, NCHUNK, CW, 2 * H)

    hc = _enc(x, enc_W, enc_b.reshape(1, H))
    for i in range(LAYERS):
        u = _sc_agg(hc, src3, dst3)
        z1, st1 = _mlp1(u, W1c[i], b1[i].reshape(1, 2 * H))
        z2, st2 = _mlp2(z1, st1, g1[i].reshape(1, 2 * H),
                        be1[i].reshape(1, 2 * H), W2[i], b2[i].reshape(1, H))
        if i == 0:
            hc = _resid0(z2, st2, g2[i].reshape(1, H), be2[i].reshape(1, H))
        else:
            hc = _resid(z2, st2, g2[i].reshape(1, H), be2[i].reshape(1, H), hc)
    ps, pc = _pool(hc, batch3)
    return _head(ps, pc, hW1, hb1.reshape(1, -1), hg1.reshape(1, -1),
                 hbe1.reshape(1, -1), hW2, hb2.reshape(1, -1),
                 hg2.reshape(1, -1), hbe2.reshape(1, -1), hW3,
                 hb3.reshape(1, -1))
